# Initial kernel scaffold; baseline (speedup 1.0000x reference)
import functools

import jax
import jax.numpy as jnp
from jax import lax
from jax.experimental import pallas as pl
from jax.experimental.pallas import tpu as pltpu
from jax.experimental.pallas import tpu_sc as plsc

VOCAB = 1000000
DIM = 64
BATCH = 1024
GROUP = 10
SEQ = 20
N_DOC = 2
N_USER = 2

SEGS = N_DOC * BATCH * GROUP + N_USER * BATCH
NW = 32
SEG_PER_W = SEGS // NW
CHUNK = 32
N_CHUNKS = SEG_PER_W // CHUNK
IDS_PER_CHUNK = CHUNK * SEQ
GATHER_SLICE = 128
N_GATHERS = IDS_PER_CHUNK // GATHER_SLICE
LANES = 16
NVREG = DIM // LANES


def _sc_body(ids_hbm, lens_hbm, table_hbm, out_hbm, idx_v, rows_v, lens_v,
             out_v, sem):
    c = lax.axis_index("c")
    s = lax.axis_index("s")
    wid = s * 2 + c
    seg_base = wid * SEG_PER_W

    pltpu.sync_copy(lens_hbm.at[pl.ds(seg_base, SEG_PER_W)], lens_v)

    def chunk_body(ci, _):
        id_base = (seg_base + ci * CHUNK) * SEQ
        pltpu.sync_copy(ids_hbm.at[pl.ds(id_base, IDS_PER_CHUNK)], idx_v)

        copies = []
        for k in range(N_GATHERS):
            cp = pltpu.async_copy(
                table_hbm.at[idx_v.at[pl.ds(k * GATHER_SLICE, GATHER_SLICE)]],
                rows_v.at[pl.ds(k * GATHER_SLICE, GATHER_SLICE), :],
                sem,
            )
            copies.append(cp)
        for cp in copies:
            cp.wait()

        def seg_body(si, _):
            ln = lens_v[ci * CHUNK + si]
            inv = jnp.where(
                ln > 0,
                1.0 / jnp.maximum(ln, 1).astype(jnp.float32),
                0.0,
            )
            accs = [jnp.zeros((LANES,), jnp.float32) for _ in range(NVREG)]
            row0 = si * SEQ
            for l in range(SEQ):
                w = jnp.where(l < ln, inv, 0.0)
                for d in range(NVREG):
                    r = rows_v[row0 + l, pl.ds(d * LANES, LANES)]
                    accs[d] = accs[d] + r * w
            for d in range(NVREG):
                out_v[pl.ds(si * DIM + d * LANES, LANES)] = accs[d]
            return ()

        lax.fori_loop(0, CHUNK, seg_body, ())
        pltpu.sync_copy(
            out_v,
            out_hbm.at[pl.ds((seg_base + ci * CHUNK) * DIM, CHUNK * DIM)],
        )
        return ()

    lax.fori_loop(0, N_CHUNKS, chunk_body, ())


@jax.jit
def _run(ids_flat, lens_flat, table):
    mesh = plsc.VectorSubcoreMesh(core_axis_name="c", subcore_axis_name="s")
    kern = functools.partial(
        pl.kernel,
        mesh=mesh,
        out_type=jax.ShapeDtypeStruct((SEGS * DIM,), jnp.float32),
        scratch_types=[
            pltpu.VMEM((IDS_PER_CHUNK,), jnp.int32),
            pltpu.VMEM((IDS_PER_CHUNK, DIM), jnp.float32),
            pltpu.VMEM((SEG_PER_W,), jnp.int32),
            pltpu.VMEM((CHUNK * DIM,), jnp.float32),
            pltpu.SemaphoreType.DMA,
        ],
    )(_sc_body)
    return kern(ids_flat, lens_flat, table)


def kernel(doc_ids, doc_len, user_ids, user_len, table):
    ids_doc = jnp.transpose(doc_ids, (1, 2, 0, 3)).reshape(-1, SEQ)
    ids_user = jnp.transpose(user_ids, (1, 0, 2)).reshape(-1, SEQ)
    ids_flat = jnp.concatenate([ids_doc, ids_user], axis=0).reshape(-1)
    len_doc = jnp.transpose(doc_len, (1, 2, 0)).reshape(-1)
    len_user = jnp.transpose(user_len, (1, 0)).reshape(-1)
    lens_flat = jnp.concatenate([len_doc, len_user], axis=0)

    out = _run(ids_flat, lens_flat, table)

    n_doc_rows = N_DOC * BATCH * GROUP
    doc_ftrs = out[: n_doc_rows * DIM].reshape(BATCH, GROUP, N_DOC, DIM)
    user_ftrs = out[n_doc_rows * DIM:].reshape(BATCH, N_USER, DIM)
    return (doc_ftrs, user_ftrs)


# speedup vs baseline: 1.0831x; 1.0831x over previous
import functools

import jax
import jax.numpy as jnp
from jax import lax
from jax.experimental import pallas as pl
from jax.experimental.pallas import tpu as pltpu
from jax.experimental.pallas import tpu_sc as plsc

VOCAB = 1000000
DIM = 64
BATCH = 1024
GROUP = 10
SEQ = 20
N_DOC = 2
N_USER = 2

SEGS = N_DOC * BATCH * GROUP + N_USER * BATCH
NW = 32
SEG_PER_W = SEGS // NW
CHUNK = 32
N_CHUNKS = SEG_PER_W // CHUNK
IDS_PER_CHUNK = CHUNK * SEQ
GATHER_SLICE = 128
N_GATHERS = IDS_PER_CHUNK // GATHER_SLICE
LANES = 16
NVREG = DIM // LANES


def _sc_body(ids_hbm, lens_hbm, table_hbm, out_hbm, idx_v, rows_v, lens_v,
             out_v, sem):
    c = lax.axis_index("c")
    s = lax.axis_index("s")
    wid = s * 2 + c
    seg_base = wid * SEG_PER_W

    pltpu.sync_copy(lens_hbm.at[pl.ds(seg_base, SEG_PER_W)],
                    lens_v.at[pl.ds(0, SEG_PER_W)])

    def chunk_body(ci, _):
        id_base = (seg_base + ci * CHUNK) * SEQ
        pltpu.sync_copy(ids_hbm.at[pl.ds(id_base, IDS_PER_CHUNK)], idx_v)

        copies = []
        for k in range(N_GATHERS):
            cp = pltpu.async_copy(
                table_hbm.at[idx_v.at[pl.ds(k * GATHER_SLICE, GATHER_SLICE)]],
                rows_v.at[pl.ds(k * GATHER_SLICE, GATHER_SLICE), :],
                sem,
            )
            copies.append(cp)
        for cp in copies:
            cp.wait()

        def seg_body(si, _):
            lv = lens_v[pl.ds(ci * CHUNK + si, LANES)]
            ln = lv[0]
            invv = 1.0 / jnp.maximum(lv.astype(jnp.float32), 1.0)
            inv = jnp.where(ln > 0, invv[0], 0.0)
            accs = [jnp.zeros((LANES,), jnp.float32) for _ in range(NVREG)]
            row0 = si * SEQ
            for l in range(SEQ):
                w = jnp.where(l < ln, inv, 0.0)
                for d in range(NVREG):
                    r = rows_v[row0 + l, pl.ds(d * LANES, LANES)]
                    accs[d] = accs[d] + r * w
            for d in range(NVREG):
                out_v[pl.ds(si * DIM + d * LANES, LANES)] = accs[d]
            return ()

        lax.fori_loop(0, CHUNK, seg_body, ())
        pltpu.sync_copy(
            out_v,
            out_hbm.at[pl.ds((seg_base + ci * CHUNK) * DIM, CHUNK * DIM)],
        )
        return ()

    lax.fori_loop(0, N_CHUNKS, chunk_body, ())


@jax.jit
def _run(ids_flat, lens_flat, table):
    mesh = plsc.VectorSubcoreMesh(core_axis_name="c", subcore_axis_name="s")
    kern = functools.partial(
        pl.kernel,
        mesh=mesh,
        out_type=jax.ShapeDtypeStruct((SEGS * DIM,), jnp.float32),
        scratch_types=[
            pltpu.VMEM((IDS_PER_CHUNK,), jnp.int32),
            pltpu.VMEM((IDS_PER_CHUNK, DIM), jnp.float32),
            pltpu.VMEM((SEG_PER_W + LANES,), jnp.int32),
            pltpu.VMEM((CHUNK * DIM,), jnp.float32),
            pltpu.SemaphoreType.DMA,
        ],
        compiler_params=pltpu.CompilerParams(use_tc_tiling_on_sc=False),
    )(_sc_body)
    return kern(ids_flat, lens_flat, table)


def kernel(doc_ids, doc_len, user_ids, user_len, table):
    ids_doc = jnp.transpose(doc_ids, (1, 2, 0, 3)).reshape(-1, SEQ)
    ids_user = jnp.transpose(user_ids, (1, 0, 2)).reshape(-1, SEQ)
    ids_flat = jnp.concatenate([ids_doc, ids_user], axis=0).reshape(-1)
    len_doc = jnp.transpose(doc_len, (1, 2, 0)).reshape(-1)
    len_user = jnp.transpose(user_len, (1, 0)).reshape(-1)
    lens_flat = jnp.concatenate([len_doc, len_user], axis=0)

    out = _run(ids_flat, lens_flat, table)

    n_doc_rows = N_DOC * BATCH * GROUP
    doc_ftrs = out[: n_doc_rows * DIM].reshape(BATCH, GROUP, N_DOC, DIM)
    user_ftrs = out[n_doc_rows * DIM:].reshape(BATCH, N_USER, DIM)
    return (doc_ftrs, user_ftrs)
